# baseline (speedup 1.0000x reference)
import functools

import jax
import jax.numpy as jnp
from jax import lax
from jax.experimental import pallas as pl
from jax.experimental.pallas import tpu as pltpu
from jax.experimental.pallas import tpu_sc as plsc

LAT = 16
BATCH = 16384
NC = 2
NS = 16
NW = NC * NS
PER_W = BATCH // NW
CH = 128
NCH = PER_W // CH


def _gmf_body(uidx_hbm, iidx_hbm, utab_hbm, itab_hbm, wb_hbm, out_hbm,
              uidx_v, iidx_v, urows_v, irows_v, wb_v, out_v, sem):
    c = lax.axis_index("c")
    s = lax.axis_index("s")
    wid = s * NC + c

    pltpu.sync_copy(uidx_hbm.at[wid], uidx_v)
    pltpu.sync_copy(iidx_hbm.at[wid], iidx_v)
    pltpu.sync_copy(wb_hbm, wb_v)

    copies = []
    for k in range(NCH):
        copies.append(pltpu.async_copy(
            utab_hbm.at[uidx_v.at[k]], urows_v.at[pl.ds(k * CH, CH)], sem))
        copies.append(pltpu.async_copy(
            itab_hbm.at[iidx_v.at[k]], irows_v.at[pl.ds(k * CH, CH)], sem))
    for cp in copies:
        cp.wait()

    wsplat = [wb_v[d, :] for d in range(LAT)]
    bvec = wb_v[LAT, :]
    iota = lax.iota(jnp.int32, LAT)

    def body(j, carry):
        ridx = j * LAT + iota
        acc = bvec
        for d in range(LAT):
            cidx = jnp.full((LAT,), d, jnp.int32)
            ucol = plsc.load_gather(urows_v, [ridx, cidx])
            icol = plsc.load_gather(irows_v, [ridx, cidx])
            acc = acc + ucol * icol * wsplat[d]
        out_v[pl.ds(j * LAT, LAT)] = acc
        return carry

    lax.fori_loop(0, PER_W // LAT, body, 0)

    pltpu.sync_copy(out_v, out_hbm.at[pl.ds(wid * PER_W, PER_W)])


_gmf = functools.partial(
    pl.kernel,
    out_type=jax.ShapeDtypeStruct((BATCH,), jnp.float32),
    mesh=plsc.VectorSubcoreMesh(core_axis_name="c", subcore_axis_name="s"),
    scratch_types=[
        pltpu.VMEM((NCH, CH), jnp.int32),
        pltpu.VMEM((NCH, CH), jnp.int32),
        pltpu.VMEM((PER_W, LAT), jnp.float32),
        pltpu.VMEM((PER_W, LAT), jnp.float32),
        pltpu.VMEM((LAT + 1, LAT), jnp.float32),
        pltpu.VMEM((PER_W,), jnp.float32),
        pltpu.SemaphoreType.DMA,
    ],
    compiler_params=pltpu.CompilerParams(
        needs_layout_passes=False, use_tc_tiling_on_sc=False),
)(_gmf_body)


@jax.jit
def kernel(user_indices, item_indices, user_table, item_table, W, b):
    uidx = user_indices.astype(jnp.int32).reshape(NW, NCH, CH)
    iidx = item_indices.astype(jnp.int32).reshape(NW, NCH, CH)
    wb = jnp.concatenate([W.reshape(LAT), b.reshape(1)])
    wb = jnp.broadcast_to(wb[:, None], (LAT + 1, LAT))
    out = _gmf(uidx, iidx, user_table, item_table, wb)
    return out.reshape(BATCH, 1)


# speedup vs baseline: 1.0031x; 1.0031x over previous
import functools

import jax
import jax.numpy as jnp
from jax import lax
from jax.experimental import pallas as pl
from jax.experimental.pallas import tpu as pltpu
from jax.experimental.pallas import tpu_sc as plsc

LAT = 16
HALF = LAT // 2
BATCH = 16384
NC = 2
NS = 16
NW = NC * NS
PER_W = BATCH // NW
CH = 128
NCH = PER_W // CH
NBLK = PER_W // LAT


def _gmf_body(uidx_hbm, iidx_hbm, utab_hbm, itab_hbm, wb_hbm, out_hbm,
              uidx_v, iidx_v, idx2_v, ubufa_v, ubufb_v, ibufa_v, ibufb_v,
              wb_v, out_v, sem):
    c = lax.axis_index("c")
    s = lax.axis_index("s")
    wid = s * NC + c

    pltpu.sync_copy(uidx_hbm.at[wid], uidx_v)
    pltpu.sync_copy(iidx_hbm.at[wid], iidx_v)
    pltpu.sync_copy(wb_hbm, wb_v)

    for k in range(NCH):
        for j in range(CH // LAT):
            sl = pl.ds(j * LAT, LAT)
            uv = uidx_v[k, sl] * 2
            iv = iidx_v[k, sl] * 2
            idx2_v[0, k, sl] = uv
            idx2_v[1, k, sl] = uv + 1
            idx2_v[2, k, sl] = iv
            idx2_v[3, k, sl] = iv + 1

    copies = []
    for k in range(NCH):
        dst = pl.ds(k * CH, CH)
        copies.append(pltpu.async_copy(
            utab_hbm.at[idx2_v.at[0, k]], ubufa_v.at[dst], sem))
        copies.append(pltpu.async_copy(
            utab_hbm.at[idx2_v.at[1, k]], ubufb_v.at[dst], sem))
        copies.append(pltpu.async_copy(
            itab_hbm.at[idx2_v.at[2, k]], ibufa_v.at[dst], sem))
        copies.append(pltpu.async_copy(
            itab_hbm.at[idx2_v.at[3, k]], ibufb_v.at[dst], sem))
    for cp in copies:
        cp.wait()

    wsplat = [wb_v[d, :] for d in range(LAT)]
    bvec = wb_v[LAT, :]
    iota = lax.iota(jnp.int32, LAT)

    def body(j, carry):
        ridx = j * LAT + iota
        acc = bvec
        for d in range(HALF):
            cidx = jnp.full((LAT,), d, jnp.int32)
            ucol = plsc.load_gather(ubufa_v, [ridx, cidx])
            icol = plsc.load_gather(ibufa_v, [ridx, cidx])
            acc = acc + ucol * icol * wsplat[d]
        for d in range(HALF):
            cidx = jnp.full((LAT,), d, jnp.int32)
            ucol = plsc.load_gather(ubufb_v, [ridx, cidx])
            icol = plsc.load_gather(ibufb_v, [ridx, cidx])
            acc = acc + ucol * icol * wsplat[HALF + d]
        out_v[pl.ds(j * LAT, LAT)] = acc
        return carry

    lax.fori_loop(0, NBLK, body, 0)

    pltpu.sync_copy(out_v, out_hbm.at[pl.ds(wid * PER_W, PER_W)])


_gmf = functools.partial(
    pl.kernel,
    out_type=jax.ShapeDtypeStruct((BATCH,), jnp.float32),
    mesh=plsc.VectorSubcoreMesh(core_axis_name="c", subcore_axis_name="s"),
    scratch_types=[
        pltpu.VMEM((NCH, CH), jnp.int32),
        pltpu.VMEM((NCH, CH), jnp.int32),
        pltpu.VMEM((4, NCH, CH), jnp.int32),
        pltpu.VMEM((PER_W, HALF), jnp.float32),
        pltpu.VMEM((PER_W, HALF), jnp.float32),
        pltpu.VMEM((PER_W, HALF), jnp.float32),
        pltpu.VMEM((PER_W, HALF), jnp.float32),
        pltpu.VMEM((LAT + 1, LAT), jnp.float32),
        pltpu.VMEM((PER_W,), jnp.float32),
        pltpu.SemaphoreType.DMA,
    ],
    compiler_params=pltpu.CompilerParams(
        needs_layout_passes=False, use_tc_tiling_on_sc=False),
)(_gmf_body)


@jax.jit
def kernel(user_indices, item_indices, user_table, item_table, W, b):
    uidx = user_indices.astype(jnp.int32).reshape(NW, NCH, CH)
    iidx = item_indices.astype(jnp.int32).reshape(NW, NCH, CH)
    utab = user_table.reshape(-1, HALF)
    itab = item_table.reshape(-1, HALF)
    wb = jnp.concatenate([W.reshape(LAT), b.reshape(1)])
    wb = jnp.broadcast_to(wb[:, None], (LAT + 1, LAT))
    out = _gmf(uidx, iidx, utab, itab, wb)
    return out.reshape(BATCH, 1)


# speedup vs baseline: 5.9546x; 5.9362x over previous
import functools

import jax
import jax.numpy as jnp
from jax import lax
from jax.experimental import pallas as pl
from jax.experimental.pallas import tpu as pltpu
from jax.experimental.pallas import tpu_sc as plsc

LAT = 16
BATCH = 16384
NC = 2
NS = 16
NW = NC * NS
PER_W = BATCH // NW
GRP = 8
NGRP = PER_W // GRP


def _gmf_body(uidx_hbm, iidx_hbm, utab_hbm, itab_hbm, wb_hbm, out_hbm,
              uidx_v, iidx_v, win_v, wb_v, out_v, sem0, sem1):
    c = lax.axis_index("c")
    s = lax.axis_index("s")
    wid = s * NC + c

    pltpu.sync_copy(uidx_hbm.at[wid], uidx_v.at[pl.ds(0, PER_W)])
    pltpu.sync_copy(iidx_hbm.at[wid], iidx_v.at[pl.ds(0, PER_W)])
    pltpu.sync_copy(wb_hbm, wb_v)

    sems = [sem0, sem1]

    def fire(g, buf):
        base = g * GRP
        uwv = (uidx_v[pl.ds(base, LAT)] >> 7) * 128
        iwv = (iidx_v[pl.ds(base, LAT)] >> 7) * 128
        for e in range(GRP):
            us = pl.multiple_of(uwv[e], 128)
            is_ = pl.multiple_of(iwv[e], 128)
            pltpu.async_copy(
                utab_hbm.at[:, pl.ds(us, 128)], win_v.at[buf, e], sems[buf])
            pltpu.async_copy(
                itab_hbm.at[:, pl.ds(is_, 128)], win_v.at[buf, GRP + e],
                sems[buf])

    def drain(buf):
        for e in range(GRP):
            pltpu.make_async_copy(
                utab_hbm.at[:, pl.ds(0, 128)], win_v.at[buf, e],
                sems[buf]).wait()
            pltpu.make_async_copy(
                utab_hbm.at[:, pl.ds(0, 128)], win_v.at[buf, GRP + e],
                sems[buf]).wait()

    lane = lax.iota(jnp.int32, LAT)
    elane = lane % GRP
    dvecs = [(lane // GRP) * 8 + dd for dd in range(GRP)]
    zeros = lane * 0
    lowm = lane < GRP
    highm = lane >= GRP
    wvs = [plsc.load_gather(wb_v, [dvecs[dd], zeros]) for dd in range(GRP)]
    bvec = wb_v[LAT, :]

    def compute(g, sbuf):
        buf = zeros + sbuf
        base = g * GRP
        epos = base + elane
        ucol = plsc.load_gather(uidx_v, [epos]) & 127
        icol = plsc.load_gather(iidx_v, [epos]) & 127
        acc = lax.full((LAT,), 0.0, jnp.float32)
        for dd in range(GRP):
            u = plsc.load_gather(win_v, [buf, elane, dvecs[dd], ucol])
            it = plsc.load_gather(
                win_v, [buf, elane + GRP, dvecs[dd], icol])
            acc = acc + u * it * wvs[dd]
        plsc.store_scatter(out_v, [epos], acc + bvec, mask=lowm)
        plsc.addupdate_scatter(out_v, [epos], acc, mask=highm)
        return ()

    fire(0, 0)

    def body(k, carry):
        g = k * 2
        fire(g + 1, 1)
        drain(0)
        compute(g, 0)
        fire(g + 2, 0)
        drain(1)
        compute(g + 1, 1)
        return carry

    lax.fori_loop(0, NGRP // 2 - 1, body, 0)
    g = NGRP - 2
    fire(g + 1, 1)
    drain(0)
    compute(g, 0)
    drain(1)
    compute(g + 1, 1)

    pltpu.sync_copy(out_v, out_hbm.at[pl.ds(wid * PER_W, PER_W)])


_gmf = functools.partial(
    pl.kernel,
    out_type=jax.ShapeDtypeStruct((BATCH,), jnp.float32),
    mesh=plsc.VectorSubcoreMesh(core_axis_name="c", subcore_axis_name="s"),
    scratch_types=[
        pltpu.VMEM((PER_W + 8,), jnp.int32),
        pltpu.VMEM((PER_W + 8,), jnp.int32),
        pltpu.VMEM((2, 2 * GRP, LAT, 128), jnp.float32),
        pltpu.VMEM((LAT + 1, LAT), jnp.float32),
        pltpu.VMEM((PER_W,), jnp.float32),
        pltpu.SemaphoreType.DMA,
        pltpu.SemaphoreType.DMA,
    ],
    compiler_params=pltpu.CompilerParams(
        needs_layout_passes=False, use_tc_tiling_on_sc=True),
)(_gmf_body)


@jax.jit
def kernel(user_indices, item_indices, user_table, item_table, W, b):
    uidx = user_indices.astype(jnp.int32).reshape(NW, PER_W)
    iidx = item_indices.astype(jnp.int32).reshape(NW, PER_W)
    wb = jnp.concatenate([W.reshape(LAT), b.reshape(1)])
    wb = jnp.broadcast_to(wb[:, None], (LAT + 1, LAT))
    out = _gmf(uidx, iidx, user_table.T, item_table.T, wb)
    return out.reshape(BATCH, 1)
